# Initial kernel scaffold; baseline (speedup 1.0000x reference)
import jax
import jax.numpy as jnp
from jax.experimental import pallas as pl


def kernel(z_where, z_present, z_what_loc, z_what_scale, z_depth_loc, z_depth_scale):
    raise NotImplementedError("write your pallas kernel here")


# speedup vs baseline: 2.4113x; 2.4113x over previous
import functools

import numpy as np
import jax
import jax.numpy as jnp
from jax import lax
from jax.experimental import pallas as pl
from jax.experimental.pallas import tpu as pltpu
from jax.experimental.pallas import tpu_sc as plsc

B = 64
N = 8192
D_WHAT = 64
MAXO = 10
N_OBJ = MAXO + int(0.1 * N)
PAD = 832
CH = 208
NCH = 4
EPS = 1e-3

_RANK_CACHE = None


def _rotl32(x, r):
    return ((x << np.uint32(r)) | (x >> np.uint32(32 - r))).astype(np.uint32)


def _threefry2x32(k0, k1, x0, x1):
    rotations = [[13, 15, 26, 6], [17, 29, 16, 24]]
    ks = [np.uint32(k0), np.uint32(k1),
          np.uint32(k0) ^ np.uint32(k1) ^ np.uint32(0x1BD11BDA)]
    x = [x0.astype(np.uint32) + ks[0], x1.astype(np.uint32) + ks[1]]
    for i in range(5):
        for r in rotations[i % 2]:
            x[0] = (x[0] + x[1]).astype(np.uint32)
            x[1] = _rotl32(x[1], r)
            x[1] = x[1] ^ x[0]
        x[0] = (x[0] + ks[(i + 1) % 3]).astype(np.uint32)
        x[1] = (x[1] + ks[(i + 2) % 3] + np.uint32(i + 1)).astype(np.uint32)
    return x[0], x[1]


def _score_rank():
    global _RANK_CACHE
    if _RANK_CACHE is None:
        total = B * N
        o0, o1 = _threefry2x32(
            np.uint32(0), np.uint32(12345),
            np.zeros(total, np.uint32), np.arange(total, dtype=np.uint32))
        bits = o0 ^ o1
        fl = ((bits >> np.uint32(9)) | np.uint32(0x3F800000)).view(np.float32)
        score = np.maximum(np.float32(0.0),
                           fl - np.float32(1.0)).reshape(B, N)
        order = np.argsort(-score, axis=1, kind="stable")
        rank = np.argsort(order, axis=1, kind="stable")
        _RANK_CACHE = rank.astype(np.int32)
    return _RANK_CACHE


def _tc_body(zp_ref, rank_ref, tag_ref, pos_ref):
    v = zp_ref[...]
    r = rank_ref[...]
    present = jnp.where(v > 0.5, 1, 0).astype(jnp.int32)
    iota = lax.broadcasted_iota(jnp.int32, (B, N), 1)

    def step(_, carry):
        active, kept, c = carry
        af = active.astype(jnp.float32)
        masked_v = v * af + (af - 1.0)
        m = jnp.max(masked_v, axis=1, keepdims=True)
        cand = active * jnp.where(v == m, 1, 0)
        selidx = jnp.max(cand * iota + (cand - 1), axis=1, keepdims=True)
        chosen = active * jnp.where(iota == selidx, 1, 0)
        rj = jnp.max(chosen * r + (chosen - 1), axis=1, keepdims=True)
        validc = jnp.where(selidx >= 0, 1, 0)
        c = c + validc * jnp.where(rj < r, 1, 0)
        kept = jnp.maximum(kept, chosen)
        active = active * (1 - chosen)
        return active, kept, c

    carry0 = (present, jnp.zeros((B, N), jnp.int32), jnp.zeros((B, N), jnp.int32))
    _, kept, c = lax.fori_loop(0, MAXO, step, carry0)
    n2 = jnp.sum(kept, axis=1, keepdims=True)
    negneed = N_OBJ - n2
    negative = (1 - kept) * jnp.where((r - c) < negneed, 1, 0)
    keep = kept + negative
    tag = (kept.astype(jnp.float32)
           - negative.astype(jnp.float32) * jnp.where(v > EPS, 1.0, 2.0))
    x = keep
    s = 1
    while s < N:
        x = x + jnp.concatenate(
            [jnp.zeros((B, s), jnp.int32), x[:, :N - s]], axis=1)
        s *= 2
    pos = x - 1
    tag_ref[...] = tag
    pos_ref[...] = pos


def _phase_a(zp2d, rank):
    return pl.pallas_call(
        _tc_body,
        out_shape=[
            jax.ShapeDtypeStruct((B, N), jnp.float32),
            jax.ShapeDtypeStruct((B, N), jnp.int32),
        ],
    )(zp2d, rank)


def _sc_body(tag_hbm, pos_hbm, where_hbm, wloc_hbm, wscale_hbm, dloc_hbm,
             dscale_hbm,
             owhat_hbm, owhere_hbm, opres_hbm, odepth_hbm,
             tag_v, pos_v, idx_v, gidx_v, ptag_v, wrow_v, dloc_v, dsc_v,
             loc_c, sc_c, owhat_c, owhere_v, opres_v, odepth_v,
             sem, sem2):
    wid = lax.axis_index("s") * 2 + lax.axis_index("c")

    def do_row(b):
        pltpu.sync_copy(tag_hbm.at[b], tag_v)
        pltpu.sync_copy(pos_hbm.at[b], pos_v)
        pltpu.sync_copy(where_hbm.at[b], wrow_v)
        pltpu.sync_copy(dloc_hbm.at[b], dloc_v)
        pltpu.sync_copy(dscale_hbm.at[b], dsc_v)
        idx_v[pl.ds(PAD - 16, 16)] = jnp.zeros((16,), jnp.int32)

        def comp(g, _):
            t16 = tag_v[pl.ds(g * 16, 16)]
            keep16 = t16 != 0.0
            p16 = pos_v[pl.ds(g * 16, 16)]
            cols = lax.iota(jnp.int32, 16) + g * 16
            plsc.store_scatter(idx_v, [p16], cols, mask=keep16)
            plsc.store_scatter(ptag_v, [p16], t16, mask=keep16)
            return 0

        lax.fori_loop(0, N // 16, comp, 0)

        def small(g, _):
            c16 = idx_v[pl.ds(g * 16, 16)]
            gidx_v[pl.ds(g * 16, 16)] = c16 + b * N
            t16 = ptag_v[pl.ds(g * 16, 16)]
            eps16 = t16 > -1.5
            opres_v[pl.ds(g * 16, 16)] = jnp.where(t16 > 0.0, 1.0, -1.0)
            dl = plsc.load_gather(dloc_v, [c16])
            dsv = plsc.load_gather(dsc_v, [c16])
            odepth_v[pl.ds(g * 16, 16)] = jnp.where(eps16, dl + dsv, 1.0)
            outp = (lax.iota(jnp.int32, 16) + g * 16) * 4
            for d in range(4):
                wv = plsc.load_gather(wrow_v, [c16 * 4 + d])
                plsc.store_scatter(owhere_v, [outp + d], wv)
            return 0

        lax.fori_loop(0, PAD // 16, small, 0)

        for c in range(NCH):
            cp = pltpu.async_copy(
                wloc_hbm.at[gidx_v.at[pl.ds(c * CH, CH)]], loc_c, sem)
            cp2 = pltpu.async_copy(
                wscale_hbm.at[gidx_v.at[pl.ds(c * CH, CH)]], sc_c, sem2)
            cp.wait()
            cp2.wait()

            def wcomp(p, _):
                tsp = plsc.load_gather(
                    ptag_v, [jnp.full((16,), c * CH, jnp.int32) + p])
                eps16 = tsp > -1.5
                for dc in range(D_WHAT // 16):
                    l16 = loc_c[p, pl.ds(dc * 16, 16)]
                    s16 = sc_c[p, pl.ds(dc * 16, 16)]
                    owhat_c[p, pl.ds(dc * 16, 16)] = jnp.where(
                        eps16, l16 + s16, 1.0)
                return 0

            lax.fori_loop(0, CH, wcomp, 0)
            nrows = N_OBJ - 3 * CH if c == NCH - 1 else CH
            pltpu.sync_copy(owhat_c.at[pl.ds(0, nrows)],
                            owhat_hbm.at[b, pl.ds(c * CH, nrows)])

        pltpu.sync_copy(opres_v, opres_hbm.at[b])
        pltpu.sync_copy(odepth_v, odepth_hbm.at[b])
        pltpu.sync_copy(owhere_v, owhere_hbm.at[b])

    for rr in range(2):
        do_row(wid * 2 + rr)


def _phase_b(tag, pos, where_flat, wloc, wscale, dloc2, dscale2):
    mesh = plsc.VectorSubcoreMesh(core_axis_name="c", subcore_axis_name="s")
    fn = functools.partial(
        pl.kernel,
        mesh=mesh,
        compiler_params=pltpu.CompilerParams(
            needs_layout_passes=False, use_tc_tiling_on_sc=False),
        out_type=[
            jax.ShapeDtypeStruct((B, N_OBJ, D_WHAT), jnp.float32),
            jax.ShapeDtypeStruct((B, 4 * PAD), jnp.float32),
            jax.ShapeDtypeStruct((B, PAD), jnp.float32),
            jax.ShapeDtypeStruct((B, PAD), jnp.float32),
        ],
        scratch_types=[
            pltpu.VMEM((N,), jnp.float32),
            pltpu.VMEM((N,), jnp.int32),
            pltpu.VMEM((PAD,), jnp.int32),
            pltpu.VMEM((PAD,), jnp.int32),
            pltpu.VMEM((PAD,), jnp.float32),
            pltpu.VMEM((4 * N,), jnp.float32),
            pltpu.VMEM((N,), jnp.float32),
            pltpu.VMEM((N,), jnp.float32),
            pltpu.VMEM((CH, D_WHAT), jnp.float32),
            pltpu.VMEM((CH, D_WHAT), jnp.float32),
            pltpu.VMEM((CH, D_WHAT), jnp.float32),
            pltpu.VMEM((4 * PAD,), jnp.float32),
            pltpu.VMEM((PAD,), jnp.float32),
            pltpu.VMEM((PAD,), jnp.float32),
            pltpu.SemaphoreType.DMA,
            pltpu.SemaphoreType.DMA,
        ],
    )(_sc_body)
    return fn(tag, pos, where_flat, wloc, wscale, dloc2, dscale2)


def kernel(z_where, z_present, z_what_loc, z_what_scale,
           z_depth_loc, z_depth_scale):
    rank = jnp.asarray(_score_rank())
    zp = z_present[:, :, 0]
    tag, pos = _phase_a(zp, rank)
    owhat, owhere, opres, odepth = _phase_b(
        tag, pos,
        z_where.reshape(B, N * 4),
        z_what_loc.reshape(B * N, D_WHAT),
        z_what_scale.reshape(B * N, D_WHAT),
        z_depth_loc[:, :, 0],
        z_depth_scale[:, :, 0],
    )
    out_where = owhere.reshape(B, PAD, 4)[:, :N_OBJ, :]
    out_pres = opres[:, :N_OBJ, None]
    out_depth = odepth[:, :N_OBJ, None]
    return (out_where, out_pres, owhat, out_depth)
